# baseline (speedup 1.0000x reference)
import jax
import jax.numpy as jnp
from jax.experimental import pallas as pl
from jax.experimental.pallas import tpu as pltpu

_DIM = 128
_SIZE = 8192
_NUM_KEYS = 8
_SCALE = 5.0


def _body(keys_ref, mem_ref, usage_ref, res_ref, uout_ref):
    k = keys_ref[0]
    mem = mem_ref[0]
    u = usage_ref[0]

    kn = 1.0 / (1e-30 + jnp.sqrt(jnp.sum(k * k, axis=1, keepdims=True)))

    sim = jax.lax.dot_general(
        k, mem, (((1,), (1,)), ((), ())), preferred_element_type=jnp.float32)

    msq = jax.lax.dot_general(
        jnp.ones((1, _DIM), jnp.float32), mem * mem,
        (((1,), (1,)), ((), ())), preferred_element_type=jnp.float32)
    mn = 1.0 / (1e-30 + jnp.sqrt(msq))

    sim = sim * (kn * _SCALE) * mn

    m = jnp.max(sim, axis=1, keepdims=True)
    e = jnp.exp(sim - m)
    att = e / jnp.sum(e, axis=1, keepdims=True)

    res_ref[0] = jax.lax.dot_general(
        att, mem, (((1,), (0,)), ((), ())), preferred_element_type=jnp.float32)
    uout_ref[0] = u + jnp.sum(att, axis=0, keepdims=True)


def kernel(keys, memory, usage):
    batch = keys.shape[0]
    usage3 = usage.reshape(batch, 1, _SIZE)
    result, new_usage = pl.pallas_call(
        _body,
        grid=(batch,),
        in_specs=[
            pl.BlockSpec((1, _NUM_KEYS, _DIM), lambda b: (b, 0, 0)),
            pl.BlockSpec((1, _SIZE, _DIM), lambda b: (b, 0, 0)),
            pl.BlockSpec((1, 1, _SIZE), lambda b: (b, 0, 0)),
        ],
        out_specs=[
            pl.BlockSpec((1, _NUM_KEYS, _DIM), lambda b: (b, 0, 0)),
            pl.BlockSpec((1, 1, _SIZE), lambda b: (b, 0, 0)),
        ],
        out_shape=[
            jax.ShapeDtypeStruct((batch, _NUM_KEYS, _DIM), jnp.float32),
            jax.ShapeDtypeStruct((batch, 1, _SIZE), jnp.float32),
        ],
        compiler_params=pltpu.CompilerParams(
            dimension_semantics=("parallel",)),
    )(keys, memory, usage3)
    return result, new_usage.reshape(batch, _SIZE)


# speedup vs baseline: 1.0187x; 1.0187x over previous
import jax
import jax.numpy as jnp
from jax.experimental import pallas as pl
from jax.experimental.pallas import tpu as pltpu

_DIM = 128
_SIZE = 8192
_NUM_KEYS = 8
_SCALE = 5.0
_NSPLIT = 2
_CHUNK = _SIZE // _NSPLIT


def _body(*refs):
    keys_ref = refs[0]
    mem_refs = refs[1:1 + _NSPLIT]
    usage_ref = refs[1 + _NSPLIT]
    res_ref = refs[2 + _NSPLIT]
    uout_ref = refs[3 + _NSPLIT]

    k = keys_ref[0]
    u = usage_ref[0]

    kn = 1.0 / (1e-30 + jnp.sqrt(jnp.sum(k * k, axis=1, keepdims=True)))
    ones_row = jnp.ones((1, _DIM), jnp.float32)

    sims = []
    for mref in mem_refs:
        mem = mref[0]
        sim = jax.lax.dot_general(
            k, mem, (((1,), (1,)), ((), ())),
            preferred_element_type=jnp.float32)
        msq = jax.lax.dot_general(
            ones_row, mem * mem, (((1,), (1,)), ((), ())),
            preferred_element_type=jnp.float32)
        mn = 1.0 / (1e-30 + jnp.sqrt(msq))
        sims.append(sim * (kn * _SCALE) * mn)

    m = sims[0].max(axis=1, keepdims=True)
    for sim in sims[1:]:
        m = jnp.maximum(m, sim.max(axis=1, keepdims=True))

    es = [jnp.exp(sim - m) for sim in sims]
    denom = es[0].sum(axis=1, keepdims=True)
    for e in es[1:]:
        denom = denom + e.sum(axis=1, keepdims=True)
    inv = 1.0 / denom

    acc = None
    for e, mref in zip(es, mem_refs):
        att = e * inv
        part = jax.lax.dot_general(
            att, mref[0], (((1,), (0,)), ((), ())),
            preferred_element_type=jnp.float32)
        acc = part if acc is None else acc + part
    res_ref[0] = acc

    for i, e in enumerate(es):
        uout_ref[0, 0:1, i * _CHUNK:(i + 1) * _CHUNK] = (
            u[:, i * _CHUNK:(i + 1) * _CHUNK]
            + jnp.sum(e * inv, axis=0, keepdims=True))


def kernel(keys, memory, usage):
    batch = keys.shape[0]
    usage3 = usage.reshape(batch, 1, _SIZE)
    mem_specs = [
        pl.BlockSpec((1, _CHUNK, _DIM), lambda b, i=i: (b, i, 0))
        for i in range(_NSPLIT)
    ]
    result, new_usage = pl.pallas_call(
        _body,
        grid=(batch,),
        in_specs=[pl.BlockSpec((1, _NUM_KEYS, _DIM), lambda b: (b, 0, 0))]
        + mem_specs
        + [pl.BlockSpec((1, 1, _SIZE), lambda b: (b, 0, 0))],
        out_specs=[
            pl.BlockSpec((1, _NUM_KEYS, _DIM), lambda b: (b, 0, 0)),
            pl.BlockSpec((1, 1, _SIZE), lambda b: (b, 0, 0)),
        ],
        out_shape=[
            jax.ShapeDtypeStruct((batch, _NUM_KEYS, _DIM), jnp.float32),
            jax.ShapeDtypeStruct((batch, 1, _SIZE), jnp.float32),
        ],
        compiler_params=pltpu.CompilerParams(
            dimension_semantics=("parallel",)),
    )(keys, *([memory] * _NSPLIT), usage3)
    return result, new_usage.reshape(batch, _SIZE)


# speedup vs baseline: 1.0408x; 1.0217x over previous
import jax
import jax.numpy as jnp
from jax.experimental import pallas as pl
from jax.experimental.pallas import tpu as pltpu

_DIM = 128
_SIZE = 8192
_NUM_KEYS = 8
_SCALE = 5.0
_NSPLIT = 2
_CHUNK = _SIZE // _NSPLIT


def _body(*refs):
    keys_ref = refs[0]
    mem_refs = refs[1:1 + _NSPLIT]
    usage_ref = refs[1 + _NSPLIT]
    res_ref = refs[2 + _NSPLIT]
    uout_ref = refs[3 + _NSPLIT]

    k = keys_ref[0]
    u = usage_ref[0]

    kn = 1.0 / (1e-30 + jnp.sqrt(jnp.sum(k * k, axis=1, keepdims=True)))
    ones_row = jnp.ones((1, _DIM), jnp.bfloat16)
    kb = k.astype(jnp.bfloat16)

    sims = []
    membs = []
    for mref in mem_refs:
        memb = mref[0].astype(jnp.bfloat16)
        membs.append(memb)
        sim = jax.lax.dot_general(
            kb, memb, (((1,), (1,)), ((), ())),
            preferred_element_type=jnp.float32)
        msq = jax.lax.dot_general(
            ones_row, memb * memb, (((1,), (1,)), ((), ())),
            preferred_element_type=jnp.float32)
        mn = 1.0 / (1e-30 + jnp.sqrt(msq))
        sims.append(sim * (kn * _SCALE) * mn)

    es = [jnp.exp(sim) for sim in sims]
    denom = es[0].sum(axis=1, keepdims=True)
    for e in es[1:]:
        denom = denom + e.sum(axis=1, keepdims=True)
    inv = 1.0 / denom

    acc = None
    for e, memb in zip(es, membs):
        att = (e * inv).astype(jnp.bfloat16)
        part = jax.lax.dot_general(
            att, memb, (((1,), (0,)), ((), ())),
            preferred_element_type=jnp.float32)
        acc = part if acc is None else acc + part
    res_ref[0] = acc

    for i, e in enumerate(es):
        uout_ref[0, 0:1, i * _CHUNK:(i + 1) * _CHUNK] = (
            u[:, i * _CHUNK:(i + 1) * _CHUNK]
            + jnp.sum(e * inv, axis=0, keepdims=True))


def kernel(keys, memory, usage):
    batch = keys.shape[0]
    usage3 = usage.reshape(batch, 1, _SIZE)
    mem_specs = [
        pl.BlockSpec((1, _CHUNK, _DIM), lambda b, i=i: (b, i, 0))
        for i in range(_NSPLIT)
    ]
    result, new_usage = pl.pallas_call(
        _body,
        grid=(batch,),
        in_specs=[pl.BlockSpec((1, _NUM_KEYS, _DIM), lambda b: (b, 0, 0))]
        + mem_specs
        + [pl.BlockSpec((1, 1, _SIZE), lambda b: (b, 0, 0))],
        out_specs=[
            pl.BlockSpec((1, _NUM_KEYS, _DIM), lambda b: (b, 0, 0)),
            pl.BlockSpec((1, 1, _SIZE), lambda b: (b, 0, 0)),
        ],
        out_shape=[
            jax.ShapeDtypeStruct((batch, _NUM_KEYS, _DIM), jnp.float32),
            jax.ShapeDtypeStruct((batch, 1, _SIZE), jnp.float32),
        ],
        compiler_params=pltpu.CompilerParams(
            dimension_semantics=("parallel",)),
    )(keys, *([memory] * _NSPLIT), usage3)
    return result, new_usage.reshape(batch, _SIZE)
